# baseline (speedup 1.0000x reference)
import functools

import jax
import jax.numpy as jnp
from jax import lax
from jax.experimental import pallas as pl
from jax.experimental.pallas import tpu as pltpu
from jax.experimental.pallas import tpu_sc as plsc

_N = 10000
_E = 320000
_F = 128
_G = 64
_NLAYER = 3

_NC = 2
_NS = 16
_NW = _NC * _NS
_CH = 128
_NCH = 80
_SEG = 40
_EPAD = _NW * _NCH * _CH
_NROWS = _N + 16
_CPT = 624
_ZR = 16

_HI = lax.Precision.HIGHEST
_MM = lax.Precision.DEFAULT


def _sc_edge_agg(h, src3, dst3):
  mesh = plsc.VectorSubcoreMesh(core_axis_name="c", subcore_axis_name="s")

  @functools.partial(
      pl.kernel,
      mesh=mesh,
      out_type=jax.ShapeDtypeStruct((_NC, _N, _F), jnp.float32),
      scratch_types=[
          pltpu.VMEM((_SEG, _CH), jnp.int32),
          pltpu.VMEM((_SEG, _CH), jnp.int32),
          pltpu.VMEM((_CH, _F), jnp.float32),
          pltpu.VMEM((_CH, _F), jnp.float32),
          pltpu.VMEM((_ZR, _F), jnp.float32),
          pltpu.VMEM_SHARED((_NROWS, _F), jnp.float32),
          pltpu.SemaphoreType.DMA,
          pltpu.SemaphoreType.DMA,
          pltpu.SemaphoreType.DMA,
          pltpu.SemaphoreType.DMA,
      ],
  )
  def sc_kernel(h_hbm, src_hbm, dst_hbm, out_hbm,
                src_v, dst_v, rows0, rows1, zero_v, acc,
                semg0, semg1, sems0, sems1):
    c = lax.axis_index("c")
    s = lax.axis_index("s")
    wid = s * _NC + c

    def zb(i, _):
      r = i // 8
      col = (i % 8) * 16
      zero_v[r, pl.ds(col, 16)] = jnp.zeros((16,), jnp.float32)
      return 0
    lax.fori_loop(0, _ZR * 8, zb, 0)

    def zc(i, _):
      pltpu.sync_copy(zero_v, acc.at[pl.ds(s * _CPT + i * _ZR, _ZR)])
      return 0
    lax.fori_loop(0, 640 // _ZR, zc, 0)

    plsc.subcore_barrier()

    def fire_g(i, buf, sem):
      pltpu.async_copy(h_hbm.at[src_v.at[i]], buf, sem)

    def wait_g(i, buf, sem):
      pltpu.make_async_copy(h_hbm.at[src_v.at[i]], buf, sem).wait()

    def fire_s(i, buf, sem):
      pltpu.async_copy(buf, acc.at[dst_v.at[i]], sem, add=True)

    def wait_s(i, buf, sem):
      pltpu.make_async_copy(buf, acc.at[dst_v.at[i]], sem).wait()

    def body(j, _):
      i = j * 2
      wait_g(i, rows0, semg0)
      fire_s(i, rows0, sems0)
      wait_g(i + 1, rows1, semg1)
      fire_s(i + 1, rows1, sems1)
      wait_s(i, rows0, sems0)
      fire_g(i + 2, rows0, semg0)
      wait_s(i + 1, rows1, sems1)
      fire_g(i + 3, rows1, semg1)
      return 0

    for seg in range(_NCH // _SEG):
      pltpu.sync_copy(src_hbm.at[wid, pl.ds(seg * _SEG, _SEG)], src_v)
      pltpu.sync_copy(dst_hbm.at[wid, pl.ds(seg * _SEG, _SEG)], dst_v)
      fire_g(0, rows0, semg0)
      fire_g(1, rows1, semg1)
      lax.fori_loop(0, _SEG // 2 - 1, body, 0)
      i = _SEG - 2
      wait_g(i, rows0, semg0)
      fire_s(i, rows0, sems0)
      wait_g(i + 1, rows1, semg1)
      fire_s(i + 1, rows1, sems1)
      wait_s(i, rows0, sems0)
      wait_s(i + 1, rows1, sems1)

    plsc.subcore_barrier()
    pltpu.sync_copy(acc.at[pl.ds(s * _CPT, _CPT)],
                    out_hbm.at[c, pl.ds(s * _CPT, _CPT)])

    @pl.when(s == _NS - 1)
    def _():
      pltpu.sync_copy(acc.at[pl.ds(_NS * _CPT, _N - _NS * _CPT)],
                      out_hbm.at[c, pl.ds(_NS * _CPT, _N - _NS * _CPT)])

  return sc_kernel(h, src3, dst3)


def _pool(mask, hn):
  ones = jnp.ones((_N, 1), jnp.float32)
  cnt = jnp.clip(lax.dot_general(mask, ones, (((0,), (0,)), ((), ())),
                                 precision=_HI), 1.0)
  ssum = lax.dot_general(mask, hn, (((0,), (0,)), ((), ())),
                         precision=_HI)
  emb = ssum / cnt
  eb = lax.dot_general(mask, emb, (((1,), (0,)), ((), ())),
                       precision=_HI)
  d = hn - eb
  s2 = lax.dot_general(mask, d * d, (((0,), (0,)), ((), ())),
                       precision=_HI)
  return emb, jnp.sqrt(s2 / cnt)


def _bn(h, g, b):
  mu = jnp.mean(h, axis=0, keepdims=True)
  d = h - mu
  var = jnp.mean(d * d, axis=0, keepdims=True)
  return g * d / jnp.sqrt(var + 1e-5) + b


def _transform_body(x_ref, wt_ref, bt_ref, g_ref, be_ref, batch_ref,
                    h_ref, emb_ref, std_ref, mask_ref):
  h = lax.dot_general(x_ref[...], wt_ref[...], (((1,), (1,)), ((), ())),
                      precision=_MM) + bt_ref[...]
  hn = _bn(h, g_ref[...], be_ref[...])
  h_ref[...] = hn
  gid = lax.broadcasted_iota(jnp.int32, (_N, _G), 1)
  mask = (batch_ref[...] == gid).astype(jnp.float32)
  mask_ref[...] = mask
  emb, std = _pool(mask, hn)
  emb_ref[...] = emb
  std_ref[...] = std


def _layer_body(h_ref, p0_ref, p1_ref, w1_ref, w2_ref, g_ref, be_ref,
                mask_ref, ho_ref, emb_ref, std_ref):
  z = h_ref[...] + p0_ref[...] + p1_ref[...]
  z = jnp.maximum(
      lax.dot_general(z, w1_ref[...], (((1,), (1,)), ((), ())),
                      precision=_MM), 0.0)
  z = lax.dot_general(z, w2_ref[...], (((1,), (1,)), ((), ())),
                      precision=_MM)
  hn = _bn(jnp.maximum(z, 0.0), g_ref[...], be_ref[...])
  ho_ref[...] = hn
  emb, std = _pool(mask_ref[...], hn)
  emb_ref[...] = emb
  std_ref[...] = std


_f32 = jnp.float32

_transform_call = pl.pallas_call(
    _transform_body,
    out_shape=[
        jax.ShapeDtypeStruct((_N, _F), _f32),
        jax.ShapeDtypeStruct((_G, _F), _f32),
        jax.ShapeDtypeStruct((_G, _F), _f32),
        jax.ShapeDtypeStruct((_N, _G), _f32),
    ],
)

_layer_call = pl.pallas_call(
    _layer_body,
    out_shape=[
        jax.ShapeDtypeStruct((_N, _F), _f32),
        jax.ShapeDtypeStruct((_G, _F), _f32),
        jax.ShapeDtypeStruct((_G, _F), _f32),
    ],
)


def kernel(x, edge_index, batch, Wt, bt, g0, beta0, W1s, W2s, gs, bs):
  src = edge_index[0]
  dst = edge_index[1]
  npad = _EPAD - _E
  pad_dst = _N + (jnp.arange(npad, dtype=jnp.int32) % 16)
  src3 = (jnp.concatenate([src, jnp.zeros((npad,), jnp.int32)])
          .reshape(_NCH, _NW, _CH).swapaxes(0, 1))
  dst3 = (jnp.concatenate([dst, pad_dst])
          .reshape(_NCH, _NW, _CH).swapaxes(0, 1))
  batch2d = batch.reshape(_N, 1)

  h, emb, std, mask = _transform_call(
      x, Wt, bt.reshape(1, _F), g0.reshape(1, _F), beta0.reshape(1, _F),
      batch2d)
  embs = [emb]
  stds = [std]
  for i in range(_NLAYER):
    p = _sc_edge_agg(h, src3, dst3)
    h, emb, std = _layer_call(
        h, p[0], p[1], W1s[i], W2s[i], gs[i].reshape(1, _F),
        bs[i].reshape(1, _F), mask)
    embs.append(emb)
    stds.append(std)
  return jnp.stack(embs), jnp.stack(stds)


# speedup vs baseline: 1.0821x; 1.0821x over previous
import functools

import jax
import jax.numpy as jnp
from jax import lax
from jax.experimental import pallas as pl
from jax.experimental.pallas import tpu as pltpu
from jax.experimental.pallas import tpu_sc as plsc

_N = 10000
_E = 320000
_F = 128
_G = 64
_NLAYER = 3

_NC = 2
_NS = 16
_NW = _NC * _NS
_CH = 64
_NCH = 160
_SEG = 40
_NBUF = 4
_EPAD = _NW * _NCH * _CH
_NROWS = _N + 16
_CPT = 624
_ZR = 16

_HI = lax.Precision.HIGHEST
_MM = lax.Precision.DEFAULT


def _sc_edge_agg(h, src3, dst3):
  mesh = plsc.VectorSubcoreMesh(core_axis_name="c", subcore_axis_name="s")

  @functools.partial(
      pl.kernel,
      mesh=mesh,
      out_type=jax.ShapeDtypeStruct((_NC, _N, _F), jnp.float32),
      scratch_types=[
          pltpu.VMEM((_SEG, _CH), jnp.int32),
          pltpu.VMEM((_SEG, _CH), jnp.int32),
          [pltpu.VMEM((_CH, _F), jnp.float32) for _ in range(_NBUF)],
          pltpu.VMEM((_ZR, _F), jnp.float32),
          pltpu.VMEM_SHARED((_NROWS, _F), jnp.float32),
          [pltpu.SemaphoreType.DMA for _ in range(_NBUF)],
      ],
  )
  def sc_kernel(h_hbm, src_hbm, dst_hbm, out_hbm,
                src_v, dst_v, rows, zero_v, acc, sems):
    c = lax.axis_index("c")
    s = lax.axis_index("s")
    wid = s * _NC + c

    def zb(i, _):
      r = i // 8
      col = (i % 8) * 16
      zero_v[r, pl.ds(col, 16)] = jnp.zeros((16,), jnp.float32)
      return 0
    lax.fori_loop(0, _ZR * 8, zb, 0)

    def zc(i, _):
      pltpu.sync_copy(zero_v, acc.at[pl.ds(s * _CPT + i * _ZR, _ZR)])
      return 0
    lax.fori_loop(0, 640 // _ZR, zc, 0)

    plsc.subcore_barrier()

    def fire_g(i, b):
      pltpu.async_copy(h_hbm.at[src_v.at[i]], rows[b], sems[b])

    def drain(i, b):
      pltpu.make_async_copy(h_hbm.at[src_v.at[i]], rows[b], sems[b]).wait()
      pltpu.sync_copy(rows[b], acc.at[dst_v.at[i]], add=True)

    def body(j, _):
      i = j * _NBUF
      for b in range(_NBUF):
        drain(i + b, b)
        fire_g(i + b + _NBUF, b)
      return 0

    for seg in range(_NCH // _SEG):
      pltpu.sync_copy(src_hbm.at[wid, pl.ds(seg * _SEG, _SEG)], src_v)
      pltpu.sync_copy(dst_hbm.at[wid, pl.ds(seg * _SEG, _SEG)], dst_v)
      for b in range(_NBUF):
        fire_g(b, b)
      lax.fori_loop(0, _SEG // _NBUF - 1, body, 0)
      for b in range(_NBUF):
        drain(_SEG - _NBUF + b, b)

    plsc.subcore_barrier()
    pltpu.sync_copy(acc.at[pl.ds(s * _CPT, _CPT)],
                    out_hbm.at[c, pl.ds(s * _CPT, _CPT)])

    @pl.when(s == _NS - 1)
    def _():
      pltpu.sync_copy(acc.at[pl.ds(_NS * _CPT, _N - _NS * _CPT)],
                      out_hbm.at[c, pl.ds(_NS * _CPT, _N - _NS * _CPT)])

  return sc_kernel(h, src3, dst3)


def _pool(mask, hn):
  ones = jnp.ones((_N, 1), jnp.float32)
  cnt = jnp.clip(lax.dot_general(mask, ones, (((0,), (0,)), ((), ())),
                                 precision=_HI), 1.0)
  ssum = lax.dot_general(mask, hn, (((0,), (0,)), ((), ())),
                         precision=_HI)
  emb = ssum / cnt
  eb = lax.dot_general(mask, emb, (((1,), (0,)), ((), ())),
                       precision=_HI)
  d = hn - eb
  s2 = lax.dot_general(mask, d * d, (((0,), (0,)), ((), ())),
                       precision=_HI)
  return emb, jnp.sqrt(s2 / cnt)


def _bn(h, g, b):
  mu = jnp.mean(h, axis=0, keepdims=True)
  d = h - mu
  var = jnp.mean(d * d, axis=0, keepdims=True)
  return g * d / jnp.sqrt(var + 1e-5) + b


def _transform_body(x_ref, wt_ref, bt_ref, g_ref, be_ref, batch_ref,
                    h_ref, emb_ref, std_ref, mask_ref):
  h = lax.dot_general(x_ref[...], wt_ref[...], (((1,), (1,)), ((), ())),
                      precision=_MM) + bt_ref[...]
  hn = _bn(h, g_ref[...], be_ref[...])
  h_ref[...] = hn
  gid = lax.broadcasted_iota(jnp.int32, (_N, _G), 1)
  mask = (batch_ref[...] == gid).astype(jnp.float32)
  mask_ref[...] = mask
  emb, std = _pool(mask, hn)
  emb_ref[...] = emb
  std_ref[...] = std


def _layer_body(h_ref, p0_ref, p1_ref, w1_ref, w2_ref, g_ref, be_ref,
                mask_ref, ho_ref, emb_ref, std_ref):
  z = h_ref[...] + p0_ref[...] + p1_ref[...]
  z = jnp.maximum(
      lax.dot_general(z, w1_ref[...], (((1,), (1,)), ((), ())),
                      precision=_MM), 0.0)
  z = lax.dot_general(z, w2_ref[...], (((1,), (1,)), ((), ())),
                      precision=_MM)
  hn = _bn(jnp.maximum(z, 0.0), g_ref[...], be_ref[...])
  ho_ref[...] = hn
  emb, std = _pool(mask_ref[...], hn)
  emb_ref[...] = emb
  std_ref[...] = std


_f32 = jnp.float32

_transform_call = pl.pallas_call(
    _transform_body,
    out_shape=[
        jax.ShapeDtypeStruct((_N, _F), _f32),
        jax.ShapeDtypeStruct((_G, _F), _f32),
        jax.ShapeDtypeStruct((_G, _F), _f32),
        jax.ShapeDtypeStruct((_N, _G), _f32),
    ],
)

_layer_call = pl.pallas_call(
    _layer_body,
    out_shape=[
        jax.ShapeDtypeStruct((_N, _F), _f32),
        jax.ShapeDtypeStruct((_G, _F), _f32),
        jax.ShapeDtypeStruct((_G, _F), _f32),
    ],
)


def kernel(x, edge_index, batch, Wt, bt, g0, beta0, W1s, W2s, gs, bs):
  src = edge_index[0]
  dst = edge_index[1]
  npad = _EPAD - _E
  pad_dst = _N + (jnp.arange(npad, dtype=jnp.int32) % 16)
  src3 = (jnp.concatenate([src, jnp.zeros((npad,), jnp.int32)])
          .reshape(_NCH, _NW, _CH).swapaxes(0, 1))
  dst3 = (jnp.concatenate([dst, pad_dst])
          .reshape(_NCH, _NW, _CH).swapaxes(0, 1))
  batch2d = batch.reshape(_N, 1)

  h, emb, std, mask = _transform_call(
      x, Wt, bt.reshape(1, _F), g0.reshape(1, _F), beta0.reshape(1, _F),
      batch2d)
  embs = [emb]
  stds = [std]
  for i in range(_NLAYER):
    p = _sc_edge_agg(h, src3, dst3)
    h, emb, std = _layer_call(
        h, p[0], p[1], W1s[i], W2s[i], gs[i].reshape(1, _F),
        bs[i].reshape(1, _F), mask)
    embs.append(emb)
    stds.append(std)
  return jnp.stack(embs), jnp.stack(stds)


# speedup vs baseline: 1.1129x; 1.0284x over previous
import functools

import jax
import jax.numpy as jnp
from jax import lax
from jax.experimental import pallas as pl
from jax.experimental.pallas import tpu as pltpu
from jax.experimental.pallas import tpu_sc as plsc

_N = 10000
_E = 320000
_F = 128
_G = 64
_NLAYER = 3

_NC = 2
_NS = 16
_NW = _NC * _NS
_CH = 64
_NCH = 160
_SEG = 40
_NBUF = 4
_EPAD = _NW * _NCH * _CH
_NROWS = _N + 16
_CPT = 624
_ZR = 16

_HI = lax.Precision.HIGHEST
_MM = lax.Precision.DEFAULT


def _sc_edge_agg(h, src3, dst3):
  mesh = plsc.VectorSubcoreMesh(core_axis_name="c", subcore_axis_name="s")

  @functools.partial(
      pl.kernel,
      mesh=mesh,
      out_type=jax.ShapeDtypeStruct((_NC, _N, _F), jnp.float32),
      scratch_types=[
          pltpu.VMEM((_SEG, _CH), jnp.int32),
          pltpu.VMEM((_SEG, _CH), jnp.int32),
          [pltpu.VMEM((_CH, _F), jnp.float32) for _ in range(_NBUF)],
          pltpu.VMEM((_ZR, _F), jnp.float32),
          pltpu.VMEM_SHARED((_NROWS, _F), jnp.float32),
          [pltpu.SemaphoreType.DMA for _ in range(_NBUF)],
      ],
  )
  def sc_kernel(h_hbm, src_hbm, dst_hbm, out_hbm,
                src_v, dst_v, rows, zero_v, acc, sems):
    c = lax.axis_index("c")
    s = lax.axis_index("s")
    wid = s * _NC + c

    def zb(i, _):
      r = i // 8
      col = (i % 8) * 16
      zero_v[r, pl.ds(col, 16)] = jnp.zeros((16,), jnp.float32)
      return 0
    lax.fori_loop(0, _ZR * 8, zb, 0)

    def zc(i, _):
      pltpu.sync_copy(zero_v, acc.at[pl.ds(s * _CPT + i * _ZR, _ZR)])
      return 0
    lax.fori_loop(0, 640 // _ZR, zc, 0)

    plsc.subcore_barrier()

    def fire_g(i, b):
      pltpu.async_copy(h_hbm.at[src_v.at[i]], rows[b], sems[b])

    def drain(i, b):
      pltpu.make_async_copy(h_hbm.at[src_v.at[i]], rows[b], sems[b]).wait()
      pltpu.sync_copy(rows[b], acc.at[dst_v.at[i]], add=True)

    def body(j, _):
      i = j * _NBUF
      for b in range(_NBUF):
        drain(i + b, b)
        fire_g(i + b + _NBUF, b)
      return 0

    for seg in range(_NCH // _SEG):
      pltpu.sync_copy(src_hbm.at[wid, pl.ds(seg * _SEG, _SEG)], src_v)
      pltpu.sync_copy(dst_hbm.at[wid, pl.ds(seg * _SEG, _SEG)], dst_v)
      for b in range(_NBUF):
        fire_g(b, b)
      lax.fori_loop(0, _SEG // _NBUF - 1, body, 0)
      for b in range(_NBUF):
        drain(_SEG - _NBUF + b, b)

    plsc.subcore_barrier()
    pltpu.sync_copy(acc.at[pl.ds(s * _CPT, _CPT)],
                    out_hbm.at[c, pl.ds(s * _CPT, _CPT)])

    @pl.when(s == _NS - 1)
    def _():
      pltpu.sync_copy(acc.at[pl.ds(_NS * _CPT, _N - _NS * _CPT)],
                      out_hbm.at[c, pl.ds(_NS * _CPT, _N - _NS * _CPT)])

  return sc_kernel(h, src3, dst3)


def _pool(mask, hn):
  ones = jnp.ones((_N, 1), jnp.float32)
  cnt = jnp.clip(lax.dot_general(mask, ones, (((0,), (0,)), ((), ())),
                                 precision=_HI), 1.0)
  ssum = lax.dot_general(mask, hn, (((0,), (0,)), ((), ())),
                         precision=_HI)
  emb = ssum / cnt
  eb = lax.dot_general(mask, emb, (((1,), (0,)), ((), ())),
                       precision=_HI)
  d = hn - eb
  s2 = lax.dot_general(mask, d * d, (((0,), (0,)), ((), ())),
                       precision=_HI)
  return emb, jnp.sqrt(s2 / cnt)


def _bn(h, g, b):
  mu = jnp.mean(h, axis=0, keepdims=True)
  d = h - mu
  var = jnp.mean(d * d, axis=0, keepdims=True)
  return g * d / jnp.sqrt(var + 1e-5) + b


def _transform_body(x_ref, wt_ref, bt_ref, g_ref, be_ref, batch_ref,
                    h_ref, mask_ref):
  h = lax.dot_general(x_ref[...], wt_ref[...], (((1,), (1,)), ((), ())),
                      precision=_MM) + bt_ref[...]
  h_ref[...] = _bn(h, g_ref[...], be_ref[...])
  gid = lax.broadcasted_iota(jnp.int32, (_N, _G), 1)
  mask_ref[...] = (batch_ref[...] == gid).astype(jnp.float32)


def _mlp_body(h_ref, p0_ref, p1_ref, w1_ref, w2_ref, g_ref, be_ref, ho_ref):
  z = h_ref[...] + p0_ref[...] + p1_ref[...]
  z = jnp.maximum(
      lax.dot_general(z, w1_ref[...], (((1,), (1,)), ((), ())),
                      precision=_MM), 0.0)
  z = lax.dot_general(z, w2_ref[...], (((1,), (1,)), ((), ())),
                      precision=_MM)
  ho_ref[...] = _bn(jnp.maximum(z, 0.0), g_ref[...], be_ref[...])


def _pool_body(h_ref, mask_ref, emb_ref, std_ref):
  emb, std = _pool(mask_ref[...], h_ref[...])
  emb_ref[...] = emb
  std_ref[...] = std


_f32 = jnp.float32

_transform_call = pl.pallas_call(
    _transform_body,
    out_shape=[
        jax.ShapeDtypeStruct((_N, _F), _f32),
        jax.ShapeDtypeStruct((_N, _G), _f32),
    ],
)

_mlp_call = pl.pallas_call(
    _mlp_body,
    out_shape=jax.ShapeDtypeStruct((_N, _F), _f32),
)

_pool_call = pl.pallas_call(
    _pool_body,
    out_shape=[
        jax.ShapeDtypeStruct((_G, _F), _f32),
        jax.ShapeDtypeStruct((_G, _F), _f32),
    ],
)


def kernel(x, edge_index, batch, Wt, bt, g0, beta0, W1s, W2s, gs, bs):
  src = edge_index[0]
  dst = edge_index[1]
  npad = _EPAD - _E
  pad_dst = _N + (jnp.arange(npad, dtype=jnp.int32) % 16)
  src3 = (jnp.concatenate([src, jnp.zeros((npad,), jnp.int32)])
          .reshape(_NCH, _NW, _CH).swapaxes(0, 1))
  dst3 = (jnp.concatenate([dst, pad_dst])
          .reshape(_NCH, _NW, _CH).swapaxes(0, 1))
  batch2d = batch.reshape(_N, 1)

  h, mask = _transform_call(
      x, Wt, bt.reshape(1, _F), g0.reshape(1, _F), beta0.reshape(1, _F),
      batch2d)
  embs = []
  stds = []
  for i in range(_NLAYER):
    p = _sc_edge_agg(h, src3, dst3)
    emb, std = _pool_call(h, mask)
    embs.append(emb)
    stds.append(std)
    h = _mlp_call(h, p[0], p[1], W1s[i], W2s[i], gs[i].reshape(1, _F),
                  bs[i].reshape(1, _F))
  emb, std = _pool_call(h, mask)
  embs.append(emb)
  stds.append(std)
  return jnp.stack(embs), jnp.stack(stds)


# speedup vs baseline: 1.1160x; 1.0028x over previous
import functools

import jax
import jax.numpy as jnp
from jax import lax
from jax.experimental import pallas as pl
from jax.experimental.pallas import tpu as pltpu
from jax.experimental.pallas import tpu_sc as plsc

_N = 10000
_E = 320000
_F = 128
_G = 64
_NLAYER = 3

_NC = 2
_NS = 16
_NW = _NC * _NS
_CH = 64
_NCH = 160
_SEG = 40
_NBUF = 4
_EPAD = _NW * _NCH * _CH
_NROWS = _N + 16
_CPT = 624
_ZR = 16

_HI = lax.Precision.HIGHEST
_MM = lax.Precision.DEFAULT


def _sc_edge_agg(h, src3, dst3):
  mesh = plsc.VectorSubcoreMesh(core_axis_name="c", subcore_axis_name="s")

  @functools.partial(
      pl.kernel,
      mesh=mesh,
      out_type=jax.ShapeDtypeStruct((_NC, _N, _F), jnp.float32),
      scratch_types=[
          pltpu.VMEM((_SEG, _CH), jnp.int32),
          pltpu.VMEM((_SEG, _CH), jnp.int32),
          [pltpu.VMEM((_CH, _F), jnp.float32) for _ in range(_NBUF)],
          pltpu.VMEM((_ZR, _F), jnp.float32),
          pltpu.VMEM_SHARED((_NROWS, _F), jnp.float32),
          [pltpu.SemaphoreType.DMA for _ in range(_NBUF)],
      ],
  )
  def sc_kernel(h_hbm, src_hbm, dst_hbm, out_hbm,
                src_v, dst_v, rows, zero_v, acc, sems):
    c = lax.axis_index("c")
    s = lax.axis_index("s")
    wid = s * _NC + c

    def fire_g(i, b):
      pltpu.async_copy(h_hbm.at[src_v.at[i]], rows[b], sems[b])

    def drain(i, b):
      pltpu.make_async_copy(h_hbm.at[src_v.at[i]], rows[b], sems[b]).wait()
      pltpu.sync_copy(rows[b], acc.at[dst_v.at[i]], add=True)

    def body(j, _):
      i = j * _NBUF
      for b in range(_NBUF):
        drain(i + b, b)
        fire_g(i + b + _NBUF, b)
      return 0

    pltpu.sync_copy(src_hbm.at[wid, pl.ds(0, _SEG)], src_v)
    pltpu.sync_copy(dst_hbm.at[wid, pl.ds(0, _SEG)], dst_v)
    for b in range(_NBUF):
      fire_g(b, b)

    def zb(i, _):
      r = i // 8
      col = (i % 8) * 16
      zero_v[r, pl.ds(col, 16)] = jnp.zeros((16,), jnp.float32)
      return 0
    lax.fori_loop(0, _ZR * 8, zb, 0)

    def zc(i, _):
      pltpu.sync_copy(zero_v, acc.at[pl.ds(s * _CPT + i * _ZR, _ZR)])
      return 0
    lax.fori_loop(0, 640 // _ZR, zc, 0)

    plsc.subcore_barrier()

    for seg in range(_NCH // _SEG):
      if seg > 0:
        pltpu.sync_copy(src_hbm.at[wid, pl.ds(seg * _SEG, _SEG)], src_v)
        pltpu.sync_copy(dst_hbm.at[wid, pl.ds(seg * _SEG, _SEG)], dst_v)
        for b in range(_NBUF):
          fire_g(b, b)
      lax.fori_loop(0, _SEG // _NBUF - 1, body, 0)
      for b in range(_NBUF):
        drain(_SEG - _NBUF + b, b)

    plsc.subcore_barrier()
    pltpu.sync_copy(acc.at[pl.ds(s * _CPT, _CPT)],
                    out_hbm.at[c, pl.ds(s * _CPT, _CPT)])

    @pl.when(s == _NS - 1)
    def _():
      pltpu.sync_copy(acc.at[pl.ds(_NS * _CPT, _N - _NS * _CPT)],
                      out_hbm.at[c, pl.ds(_NS * _CPT, _N - _NS * _CPT)])

  return sc_kernel(h, src3, dst3)


def _pool(mask, hn):
  ones = jnp.ones((_N, 1), jnp.float32)
  cnt = jnp.clip(lax.dot_general(mask, ones, (((0,), (0,)), ((), ())),
                                 precision=_HI), 1.0)
  ssum = lax.dot_general(mask, hn, (((0,), (0,)), ((), ())),
                         precision=_HI)
  emb = ssum / cnt
  eb = lax.dot_general(mask, emb, (((1,), (0,)), ((), ())),
                       precision=_HI)
  d = hn - eb
  s2 = lax.dot_general(mask, d * d, (((0,), (0,)), ((), ())),
                       precision=_HI)
  return emb, jnp.sqrt(s2 / cnt)


def _bn(h, g, b):
  mu = jnp.mean(h, axis=0, keepdims=True)
  d = h - mu
  var = jnp.mean(d * d, axis=0, keepdims=True)
  return g * d / jnp.sqrt(var + 1e-5) + b


def _transform_body(x_ref, wt_ref, bt_ref, g_ref, be_ref, batch_ref,
                    h_ref, mask_ref):
  h = lax.dot_general(x_ref[...], wt_ref[...], (((1,), (1,)), ((), ())),
                      precision=_MM) + bt_ref[...]
  h_ref[...] = _bn(h, g_ref[...], be_ref[...])
  gid = lax.broadcasted_iota(jnp.int32, (_N, _G), 1)
  mask_ref[...] = (batch_ref[...] == gid).astype(jnp.float32)


def _mlp_body(h_ref, p0_ref, p1_ref, w1_ref, w2_ref, g_ref, be_ref, ho_ref):
  z = h_ref[...] + p0_ref[...] + p1_ref[...]
  z = jnp.maximum(
      lax.dot_general(z, w1_ref[...], (((1,), (1,)), ((), ())),
                      precision=_MM), 0.0)
  z = lax.dot_general(z, w2_ref[...], (((1,), (1,)), ((), ())),
                      precision=_MM)
  ho_ref[...] = _bn(jnp.maximum(z, 0.0), g_ref[...], be_ref[...])


def _pool_body(h_ref, mask_ref, emb_ref, std_ref):
  emb, std = _pool(mask_ref[...], h_ref[...])
  emb_ref[...] = emb
  std_ref[...] = std


_f32 = jnp.float32

_transform_call = pl.pallas_call(
    _transform_body,
    out_shape=[
        jax.ShapeDtypeStruct((_N, _F), _f32),
        jax.ShapeDtypeStruct((_N, _G), _f32),
    ],
)

_mlp_call = pl.pallas_call(
    _mlp_body,
    out_shape=jax.ShapeDtypeStruct((_N, _F), _f32),
)

_pool_call = pl.pallas_call(
    _pool_body,
    out_shape=[
        jax.ShapeDtypeStruct((_G, _F), _f32),
        jax.ShapeDtypeStruct((_G, _F), _f32),
    ],
)


def kernel(x, edge_index, batch, Wt, bt, g0, beta0, W1s, W2s, gs, bs):
  src = edge_index[0]
  dst = edge_index[1]
  npad = _EPAD - _E
  pad_dst = _N + (jnp.arange(npad, dtype=jnp.int32) % 16)
  src3 = (jnp.concatenate([src, jnp.zeros((npad,), jnp.int32)])
          .reshape(_NCH, _NW, _CH).swapaxes(0, 1))
  dst3 = (jnp.concatenate([dst, pad_dst])
          .reshape(_NCH, _NW, _CH).swapaxes(0, 1))
  batch2d = batch.reshape(_N, 1)

  h, mask = _transform_call(
      x, Wt, bt.reshape(1, _F), g0.reshape(1, _F), beta0.reshape(1, _F),
      batch2d)
  embs = []
  stds = []
  for i in range(_NLAYER):
    p = _sc_edge_agg(h, src3, dst3)
    emb, std = _pool_call(h, mask)
    embs.append(emb)
    stds.append(std)
    h = _mlp_call(h, p[0], p[1], W1s[i], W2s[i], gs[i].reshape(1, _F),
                  bs[i].reshape(1, _F))
  emb, std = _pool_call(h, mask)
  embs.append(emb)
  stds.append(std)
  return jnp.stack(embs), jnp.stack(stds)
